# baseline (speedup 1.0000x reference)
import functools

import jax
import jax.numpy as jnp
from jax import lax
from jax.experimental import pallas as pl
from jax.experimental.pallas import tpu as pltpu
from jax.experimental.pallas import tpu_sc as plsc

_ROWS = 16384
_FEATS = 128
_CHUNK = 64
_NCHUNKS = _ROWS // (2 * 16) // _CHUNK


def _sc_body(
    x_hbm, aidx_hbm, a_hbm, o_hbm, xbuf0, xbuf1, obuf0, obuf1, aidx_v, a_v, insem, outsem
):
    nc = 2
    ns = 16
    wid = lax.axis_index("s") * nc + lax.axis_index("c")
    rows_per_w = _ROWS // (nc * ns)
    base = wid * rows_per_w
    cw = _CHUNK * _FEATS

    pltpu.sync_copy(aidx_hbm, aidx_v)
    pltpu.sync_copy(a_hbm, a_v)

    xbufs = (xbuf0, xbuf1)
    obufs = (obuf0, obuf1)

    def get_in(kc):
        return pltpu.make_async_copy(
            x_hbm.at[pl.ds((base + kc * _CHUNK) * _FEATS, cw)],
            xbufs[kc % 2],
            insem.at[kc % 2],
        )

    def put_out(kc):
        return pltpu.make_async_copy(
            obufs[kc % 2],
            o_hbm.at[pl.ds((base + kc * _CHUNK) * _FEATS, cw)],
            outsem.at[kc % 2],
        )

    get_in(0).start()
    for kc in range(_NCHUNKS):
        if kc + 1 < _NCHUNKS:
            get_in(kc + 1).start()
        get_in(kc).wait()
        if kc >= 2:
            put_out(kc - 2).wait()
        xbuf = xbufs[kc % 2]
        obuf = obufs[kc % 2]

        def row_body(r, carry2, xbuf=xbuf, obuf=obuf):
            for j in range(_FEATS // 16):
                idx = aidx_v[pl.ds(j * 16, 16)] + r * _FEATS
                g = plsc.load_gather(xbuf, [idx])
                z = g - a_v[pl.ds(j * 16, 16)]
                y = 1.0 / (1.0 + jnp.exp(-z))
                obuf[pl.ds(r * _FEATS + j * 16, 16)] = y
            return carry2

        lax.fori_loop(0, _CHUNK, row_body, 0)
        put_out(kc).start()
    if _NCHUNKS >= 2:
        put_out(_NCHUNKS - 2).wait()
    put_out(_NCHUNKS - 1).wait()


@jax.jit
def kernel(x, a, a_index):
    n, d = x.shape
    mesh = plsc.VectorSubcoreMesh(core_axis_name="c", subcore_axis_name="s")
    k = functools.partial(
        pl.kernel,
        mesh=mesh,
        compiler_params=pltpu.CompilerParams(needs_layout_passes=False),
        out_type=jax.ShapeDtypeStruct((n * d,), x.dtype),
        scratch_types=[
            pltpu.VMEM((_CHUNK * d,), x.dtype),
            pltpu.VMEM((_CHUNK * d,), x.dtype),
            pltpu.VMEM((_CHUNK * d,), x.dtype),
            pltpu.VMEM((_CHUNK * d,), x.dtype),
            pltpu.VMEM((d,), jnp.int32),
            pltpu.VMEM((d,), x.dtype),
            pltpu.SemaphoreType.DMA((2,)),
            pltpu.SemaphoreType.DMA((2,)),
        ],
    )(_sc_body)
    return k(x.reshape(n * d), a_index, a.reshape(d)).reshape(n, d)


# speedup vs baseline: 3.2545x; 3.2545x over previous
import functools

import jax
import jax.numpy as jnp
from jax import lax
from jax.experimental import pallas as pl
from jax.experimental.pallas import tpu as pltpu
from jax.experimental.pallas import tpu_sc as plsc

_ROWS = 16384
_FEATS = 128
_CHUNK = 128
_NCHUNKS = _ROWS // (2 * 16) // _CHUNK
_UNROLL = 8


def _sc_body(
    x_hbm, aidx_hbm, a_hbm, o_hbm, xbuf0, xbuf1, obuf0, obuf1, aidx_v, a_v, insem, outsem
):
    nc = 2
    ns = 16
    wid = lax.axis_index("s") * nc + lax.axis_index("c")
    rows_per_w = _ROWS // (nc * ns)
    base = wid * rows_per_w
    cw = _CHUNK * _FEATS

    pltpu.sync_copy(aidx_hbm, aidx_v)
    pltpu.sync_copy(a_hbm, a_v)

    xbufs = (xbuf0, xbuf1)
    obufs = (obuf0, obuf1)

    def get_in(kc):
        return pltpu.make_async_copy(
            x_hbm.at[pl.ds((base + kc * _CHUNK) * _FEATS, cw)],
            xbufs[kc % 2],
            insem.at[kc % 2],
        )

    def put_out(kc):
        return pltpu.make_async_copy(
            obufs[kc % 2],
            o_hbm.at[pl.ds((base + kc * _CHUNK) * _FEATS, cw)],
            outsem.at[kc % 2],
        )

    get_in(0).start()
    for kc in range(_NCHUNKS):
        if kc + 1 < _NCHUNKS:
            get_in(kc + 1).start()
        get_in(kc).wait()
        if kc >= 2:
            put_out(kc - 2).wait()
        xbuf = xbufs[kc % 2]
        obuf = obufs[kc % 2]

        for j in range(_FEATS // 16):
            idx0 = aidx_v[pl.ds(j * 16, 16)]
            a_j = a_v[pl.ds(j * 16, 16)]

            @plsc.parallel_loop(0, _CHUNK, 1, unroll=_UNROLL)
            def row_body(r, xbuf=xbuf, obuf=obuf, idx0=idx0, a_j=a_j, j=j):
                g = plsc.load_gather(xbuf, [idx0 + r * _FEATS])
                z = g - a_j
                y = 1.0 / (1.0 + jnp.exp(-z))
                obuf[pl.ds(r * _FEATS + j * 16, 16)] = y
        put_out(kc).start()
    if _NCHUNKS >= 2:
        put_out(_NCHUNKS - 2).wait()
    put_out(_NCHUNKS - 1).wait()


@jax.jit
def kernel(x, a, a_index):
    n, d = x.shape
    mesh = plsc.VectorSubcoreMesh(core_axis_name="c", subcore_axis_name="s")
    k = functools.partial(
        pl.kernel,
        mesh=mesh,
        compiler_params=pltpu.CompilerParams(needs_layout_passes=False),
        out_type=jax.ShapeDtypeStruct((n * d,), x.dtype),
        scratch_types=[
            pltpu.VMEM((_CHUNK * d,), x.dtype),
            pltpu.VMEM((_CHUNK * d,), x.dtype),
            pltpu.VMEM((_CHUNK * d,), x.dtype),
            pltpu.VMEM((_CHUNK * d,), x.dtype),
            pltpu.VMEM((d,), jnp.int32),
            pltpu.VMEM((d,), x.dtype),
            pltpu.SemaphoreType.DMA((2,)),
            pltpu.SemaphoreType.DMA((2,)),
        ],
    )(_sc_body)
    return k(x.reshape(n * d), a_index, a.reshape(d)).reshape(n, d)


# speedup vs baseline: 3.4451x; 1.0586x over previous
import functools

import jax
import jax.numpy as jnp
from jax import lax
from jax.experimental import pallas as pl
from jax.experimental.pallas import tpu as pltpu
from jax.experimental.pallas import tpu_sc as plsc

_ROWS = 16384
_FEATS = 128
_SC_ROWS = 3072
_SC_CHUNK = 96
_SC_NCHUNKS = _SC_ROWS // (2 * 16) // _SC_CHUNK
_UNROLL = 8
_TC_CHUNK = (_ROWS - _SC_ROWS) // 4
_NBUF = 4


def _tc_body(i_ref, a_ref, x_hbm, o_hbm, xbuf, obuf, insem, outsem):
    n = x_hbm.shape[0]
    c = _TC_CHUNK
    nchunks = n // c

    def get_in(k, slot):
        return pltpu.make_async_copy(
            x_hbm.at[pl.ds(k * c, c), :], xbuf.at[slot], insem.at[slot]
        )

    def put_out(k, slot):
        return pltpu.make_async_copy(
            obuf.at[slot], o_hbm.at[pl.ds(k * c, c), :], outsem.at[slot]
        )

    nbuf = _NBUF
    for k in range(min(nbuf - 1, nchunks)):
        get_in(k, k % nbuf).start()
    for k in range(nchunks):
        slot = k % nbuf
        if k + nbuf - 1 < nchunks:
            get_in(k + nbuf - 1, (k + nbuf - 1) % nbuf).start()
        get_in(k, slot).wait()
        if k >= nbuf:
            put_out(k - nbuf, slot).wait()
        x = xbuf[slot]
        idx = jnp.broadcast_to(i_ref[0:1, :], x.shape)
        z = jnp.take_along_axis(x, idx, axis=1) - a_ref[0:1, :]
        obuf[slot] = jax.nn.sigmoid(z)
        put_out(k, slot).start()
    for k in range(max(0, nchunks - nbuf), nchunks):
        put_out(k, k % nbuf).wait()


def _tc_part(x, idx_b, a_b):
    n, d = x.shape
    return pl.pallas_call(
        _tc_body,
        in_specs=[
            pl.BlockSpec((8, d), lambda: (0, 0)),
            pl.BlockSpec((8, d), lambda: (0, 0)),
            pl.BlockSpec(memory_space=pl.ANY),
        ],
        out_specs=pl.BlockSpec(memory_space=pl.ANY),
        out_shape=jax.ShapeDtypeStruct((n, d), x.dtype),
        scratch_shapes=[
            pltpu.VMEM((_NBUF, _TC_CHUNK, d), x.dtype),
            pltpu.VMEM((_NBUF, _TC_CHUNK, d), x.dtype),
            pltpu.SemaphoreType.DMA((_NBUF,)),
            pltpu.SemaphoreType.DMA((_NBUF,)),
        ],
    )(idx_b, a_b, x)


def _sc_body(
    x_hbm, aidx_hbm, a_hbm, o_hbm, xbuf0, xbuf1, obuf0, obuf1, aidx_v, a_v,
    insem, outsem,
):
    nc = 2
    ns = 16
    wid = lax.axis_index("s") * nc + lax.axis_index("c")
    rows_per_w = _SC_ROWS // (nc * ns)
    base = wid * rows_per_w
    cw = _SC_CHUNK * _FEATS

    pltpu.sync_copy(aidx_hbm, aidx_v)
    pltpu.sync_copy(a_hbm, a_v)

    xbufs = (xbuf0, xbuf1)
    obufs = (obuf0, obuf1)

    def get_in(kc):
        return pltpu.make_async_copy(
            x_hbm.at[pl.ds((base + kc * _SC_CHUNK) * _FEATS, cw)],
            xbufs[kc % 2],
            insem.at[kc % 2],
        )

    def put_out(kc):
        return pltpu.make_async_copy(
            obufs[kc % 2],
            o_hbm.at[pl.ds((base + kc * _SC_CHUNK) * _FEATS, cw)],
            outsem.at[kc % 2],
        )

    get_in(0).start()
    for kc in range(_SC_NCHUNKS):
        if kc + 1 < _SC_NCHUNKS:
            get_in(kc + 1).start()
        get_in(kc).wait()
        if kc >= 2:
            put_out(kc - 2).wait()
        xbuf = xbufs[kc % 2]
        obuf = obufs[kc % 2]

        for j in range(_FEATS // 16):
            idx0 = aidx_v[pl.ds(j * 16, 16)]
            a_j = a_v[pl.ds(j * 16, 16)]

            @plsc.parallel_loop(0, _SC_CHUNK, 1, unroll=_UNROLL)
            def row_body(r, xbuf=xbuf, obuf=obuf, idx0=idx0, a_j=a_j, j=j):
                g = plsc.load_gather(xbuf, [idx0 + r * _FEATS])
                z = g - a_j
                y = 1.0 / (1.0 + jnp.exp(-z))
                obuf[pl.ds(r * _FEATS + j * 16, 16)] = y

        put_out(kc).start()
    if _SC_NCHUNKS >= 2:
        put_out(_SC_NCHUNKS - 2).wait()
    put_out(_SC_NCHUNKS - 1).wait()


def _sc_part(x_flat, a_index, a_flat):
    mesh = plsc.VectorSubcoreMesh(core_axis_name="c", subcore_axis_name="s")
    k = functools.partial(
        pl.kernel,
        mesh=mesh,
        compiler_params=pltpu.CompilerParams(needs_layout_passes=False),
        out_type=jax.ShapeDtypeStruct((_SC_ROWS * _FEATS,), x_flat.dtype),
        scratch_types=[
            pltpu.VMEM((_SC_CHUNK * _FEATS,), x_flat.dtype),
            pltpu.VMEM((_SC_CHUNK * _FEATS,), x_flat.dtype),
            pltpu.VMEM((_SC_CHUNK * _FEATS,), x_flat.dtype),
            pltpu.VMEM((_SC_CHUNK * _FEATS,), x_flat.dtype),
            pltpu.VMEM((_FEATS,), jnp.int32),
            pltpu.VMEM((_FEATS,), x_flat.dtype),
            pltpu.SemaphoreType.DMA((2,)),
            pltpu.SemaphoreType.DMA((2,)),
        ],
    )(_sc_body)
    return k(x_flat, a_index, a_flat)


@jax.jit
def kernel(x, a, a_index):
    n, d = x.shape
    idx_b = jnp.broadcast_to(a_index[None, :], (8, d))
    a_b = jnp.broadcast_to(a, (8, d))
    tc_rows = n - _SC_ROWS
    y_sc = _sc_part(x[tc_rows:].reshape(_SC_ROWS * d), a_index, a.reshape(d))
    y_tc = _tc_part(x[:tc_rows], idx_b, a_b)
    return jnp.concatenate([y_tc, y_sc.reshape(_SC_ROWS, d)], axis=0)
